# Initial kernel scaffold; baseline (speedup 1.0000x reference)
import jax
import jax.numpy as jnp
from jax.experimental import pallas as pl


def kernel(input_ids, tables):
    raise NotImplementedError("write your pallas kernel here")


# speedup vs baseline: 6.0889x; 6.0889x over previous
import functools

import jax
import jax.numpy as jnp
from jax import lax
from jax.experimental import pallas as pl
from jax.experimental.pallas import tpu as pltpu
from jax.experimental.pallas import tpu_sc as plsc

_K = 4
_V = 100000
_D = 64
_B = 4096
_S = 200
_N = _B * _S
_NW = 32
_PER_W = _N // _NW
_BLK = 512
_NBLK = _PER_W // _BLK
_OUT_ROWS = _N // _K


def _make_kernel():
    mesh = plsc.VectorSubcoreMesh(core_axis_name="c", subcore_axis_name="s")

    @functools.partial(
        pl.kernel,
        mesh=mesh,
        out_type=jax.ShapeDtypeStruct((_OUT_ROWS, _D), jnp.float32),
        compiler_params=pltpu.CompilerParams(use_tc_tiling_on_sc=False),
        scratch_types=[
            pltpu.VMEM((4, 128), jnp.int32),
            pltpu.VMEM((_BLK, _D), jnp.float32),
            pltpu.VMEM((128, _D), jnp.float32),
            pltpu.SemaphoreType.DMA,
        ],
    )
    def body(ids_hbm, table_hbm, out_hbm, idx_v, rows_v, out_v, sem):
        wid = lax.axis_index("c") * 16 + lax.axis_index("s")
        offv = (lax.iota(jnp.int32, 16) % _K) * _V

        def block_body(g, carry):
            row_base = wid * (_PER_W // 128) + g * 4
            pltpu.sync_copy(ids_hbm.at[pl.ds(row_base, 4)], idx_v)
            for s in range(4):
                for c in range(8):
                    sl = pl.ds(c * 16, 16)
                    idx_v[s, sl] = idx_v[s, sl] + offv
            copies = []
            for s in range(4):
                copies.append(
                    pltpu.async_copy(
                        table_hbm.at[idx_v.at[s]],
                        rows_v.at[pl.ds(s * 128, 128)],
                        sem,
                    )
                )
            for cp in copies:
                cp.wait()

            def out_row(j, c2):
                r = j * 4
                for c in range(4):
                    sl = pl.ds(c * 16, 16)
                    acc = rows_v[r, sl]
                    acc = acc + rows_v[r + 1, sl]
                    acc = acc + rows_v[r + 2, sl]
                    acc = acc + rows_v[r + 3, sl]
                    out_v[j, sl] = acc
                return c2

            lax.fori_loop(0, 128, out_row, 0)
            out_base = wid * (_PER_W // _K) + g * 128
            pltpu.sync_copy(out_v, out_hbm.at[pl.ds(out_base, 128)])
            return carry

        lax.fori_loop(0, _NBLK, block_body, 0)

    return body


_sc_kernel = _make_kernel()


@jax.jit
def kernel(input_ids, tables):
    ids2d = input_ids.reshape(_N // 128, 128)
    table_flat = tables.reshape(_K * _V, _D)
    out = _sc_kernel(ids2d, table_flat)
    return out.reshape(_B, _S // _K, _D)


# speedup vs baseline: 9.2719x; 1.5228x over previous
import functools

import jax
import jax.numpy as jnp
from jax import lax
from jax.experimental import pallas as pl
from jax.experimental.pallas import tpu as pltpu
from jax.experimental.pallas import tpu_sc as plsc

_K = 4
_V = 100000
_D = 64
_B = 4096
_S = 200
_N = _B * _S
_NW = 32
_PER_W = _N // _NW
_IDX_ROWS = _PER_W // 128
_BLK = 512
_NBLK = _PER_W // _BLK
_OUT_BLK = _BLK // _K
_OUT_ROWS = _N // _K


def _make_kernel():
    mesh = plsc.VectorSubcoreMesh(core_axis_name="c", subcore_axis_name="s")

    @functools.partial(
        pl.kernel,
        mesh=mesh,
        out_type=jax.ShapeDtypeStruct((_OUT_ROWS, _D), jnp.float32),
        compiler_params=pltpu.CompilerParams(use_tc_tiling_on_sc=False),
        scratch_types=[
            pltpu.VMEM((_IDX_ROWS, 128), jnp.int32),
            pltpu.VMEM((2, _BLK, _D), jnp.float32),
            pltpu.VMEM((2, _OUT_BLK, _D), jnp.float32),
            pltpu.SemaphoreType.DMA,
            pltpu.SemaphoreType.DMA,
            pltpu.SemaphoreType.DMA,
            pltpu.SemaphoreType.DMA,
        ],
    )
    def body(ids_hbm, table_hbm, out_hbm, idx_v, rows_v, out_v, sg0, sg1, so0, so1):
        wid = lax.axis_index("c") * 16 + lax.axis_index("s")
        sgs = (sg0, sg1)
        sos = (so0, so1)

        pltpu.sync_copy(ids_hbm.at[pl.ds(wid * _IDX_ROWS, _IDX_ROWS)], idx_v)
        offv = (lax.iota(jnp.int32, 16) % _K) * _V

        @plsc.parallel_loop(0, _IDX_ROWS, unroll=2)
        def _(r):
            for c in range(8):
                sl = pl.ds(c * 16, 16)
                idx_v[r, sl] = idx_v[r, sl] + offv

        def fire_gathers(g, buf):
            for s in range(4):
                pltpu.async_copy(
                    table_hbm.at[idx_v.at[g * 4 + s]],
                    rows_v.at[buf].at[pl.ds(s * 128, 128)],
                    sgs[buf],
                )

        def wait_gathers(g, buf):
            for s in range(4):
                pltpu.make_async_copy(
                    table_hbm.at[idx_v.at[g * 4 + s]],
                    rows_v.at[buf].at[pl.ds(s * 128, 128)],
                    sgs[buf],
                ).wait()

        def out_slice(g):
            return out_hbm.at[pl.ds(wid * (_PER_W // _K) + g * _OUT_BLK, _OUT_BLK)]

        fire_gathers(0, 0)

        def outer(gg, carry):
            for b in range(2):
                g = gg * 2 + b

                @pl.when(g + 1 < _NBLK)
                def _():
                    fire_gathers(g + 1, 1 - b)

                wait_gathers(g, b)

                @pl.when(g >= 2)
                def _():
                    pltpu.make_async_copy(out_v.at[b], out_slice(g - 2), sos[b]).wait()

                rv = rows_v.at[b]
                ov = out_v.at[b]

                @plsc.parallel_loop(0, _OUT_BLK, unroll=4)
                def _(j):
                    r = j * 4
                    for c in range(4):
                        sl = pl.ds(c * 16, 16)
                        ov[j, sl] = (
                            rv[r, sl] + rv[r + 1, sl] + rv[r + 2, sl] + rv[r + 3, sl]
                        )

                pltpu.async_copy(out_v.at[b], out_slice(g), sos[b])
            return carry

        lax.fori_loop(0, _NBLK // 2, outer, 0)

        for b in range(2):
            pltpu.make_async_copy(out_v.at[b], out_slice(_NBLK - 2 + b), sos[b]).wait()

    return body


_sc_kernel = _make_kernel()


@jax.jit
def kernel(input_ids, tables):
    ids2d = input_ids.reshape(_N // 128, 128)
    table_flat = tables.reshape(_K * _V, _D)
    out = _sc_kernel(ids2d, table_flat)
    return out.reshape(_B, _S // _K, _D)
